# baseline (speedup 1.0000x reference)
import functools

import jax
import jax.numpy as jnp
from jax import lax
from jax.experimental import pallas as pl
from jax.experimental.pallas import tpu as pltpu
from jax.experimental.pallas import tpu_sc as plsc

D = 32
NC, NS = 2, 16
NW = NC * NS
BB = 128
S = 5
G = 3
LANES = 16
TV = 1000


def _emb_body(l_total, wordsT_hbm, tagsT_hbm, wt_hbm, ttT_hbm, out_hbm,
              widx, tidx, wrows, tagv, pbuf, sem_g, sem_i, sem_w):
    w = lax.axis_index("s") * NC + lax.axis_index("c")

    pltpu.sync_copy(wordsT_hbm.at[:, pl.ds(w * BB, BB)], widx)
    pltpu.sync_copy(ttT_hbm, tagv)

    def gathers(b, l, start):
        cp = pltpu.make_async_copy(wt_hbm.at[widx.at[l]], wrows.at[b],
                                   sem_g.at[b])
        ci = pltpu.make_async_copy(tagsT_hbm.at[l, pl.ds(w * BB, BB)],
                                   tidx.at[b], sem_i.at[b])
        if start:
            cp.start()
            ci.start()
        else:
            cp.wait()
            ci.wait()

    def write(b, l, start):
        cp = pltpu.make_async_copy(pbuf.at[b], out_hbm.at[l, :, w],
                                   sem_w.at[b])
        cp.start() if start else cp.wait()

    def transpose(b, l):
        iot = lax.iota(jnp.int32, LANES)
        src = wrows.at[b]
        for cg in range(D // LANES):

            @plsc.parallel_loop(0, LANES, unroll=2)
            def _(r):
                t = (iot + r) & (LANES - 1)
                csrc = t + cg * LANES
                ct = (t >> 3) + cg * 2
                ci = t & 7
                for bg in range(BB // LANES):
                    bvec = iot + bg * LANES
                    vals = plsc.load_gather(src, [bvec, csrc])
                    plsc.store_scatter(pbuf.at[b], [ct, ci, bvec], vals)

        tvs = [tidx[b, pl.ds(bg * LANES, LANES)]
               for bg in range(BB // LANES)]

        @plsc.parallel_loop(0, D, unroll=2)
        def _(c):
            csp = jnp.full((LANES,), 0, jnp.int32) + c
            ct = (c >> 3) + 4
            ci = c & 7
            for bg in range(BB // LANES):
                vals = plsc.load_gather(tagv, [csp, tvs[bg]])
                pbuf[b, ct, ci, pl.ds(bg * LANES, LANES)] = vals

    @pl.loop(0, l_total // S)
    def _(g):
        for u in range(S):
            l = g * S + u

            @pl.when(g > 0)
            def _():
                write(u, l - S, False)

            gathers(u, l, True)

            up = (u - G) % S

            @pl.when(l >= G)
            def _():
                gathers(up, l - G, False)
                transpose(up, l - G)
                write(up, l - G, True)

    for t in range(G):
        l = l_total - G + t
        gathers(l % S, l, False)
        transpose(l % S, l)
        write(l % S, l, True)
    for u in range(S):
        write(u, 0, False)


def _build(nb, l_total):
    assert nb == NW * BB and l_total % S == 0
    mesh = plsc.VectorSubcoreMesh(core_axis_name="c", subcore_axis_name="s")
    return functools.partial(
        pl.kernel,
        out_type=jax.ShapeDtypeStruct((l_total, 2 * D // 8, NW, 8, BB),
                                      jnp.float32),
        mesh=mesh,
        compiler_params=pltpu.CompilerParams(use_tc_tiling_on_sc=False,
                                             needs_layout_passes=False),
        scratch_types=[
            pltpu.VMEM((l_total, BB), jnp.int32),
            pltpu.VMEM((S, BB), jnp.int32),
            pltpu.VMEM((S, BB, D), jnp.float32),
            pltpu.VMEM((D, TV), jnp.float32),
            pltpu.VMEM((S, 2 * D // 8, 8, BB), jnp.float32),
            pltpu.SemaphoreType.DMA((S,)),
            pltpu.SemaphoreType.DMA((S,)),
            pltpu.SemaphoreType.DMA((S,)),
        ],
    )(functools.partial(_emb_body, l_total))


ST = 31232
CW = 256
NCH1 = 124
S1 = 4
G1 = 3


def _tr_body(v, wt_hbm, tail_hbm, out_hbm, ibuf, obuf, sem_r, sem_w):
    w = lax.axis_index("s") * NC + lax.axis_index("c")
    iot = lax.iota(jnp.int32, LANES)
    iot32 = iot * D

    def read(b, k, start):
        i0 = pl.multiple_of(w * ST + k * CW, CW)
        cp = pltpu.make_async_copy(wt_hbm.at[:, pl.ds(i0, CW)], ibuf.at[b],
                                   sem_r.at[b])
        cp.start() if start else cp.wait()

    def write(b, k, start):
        r0 = pl.multiple_of((w * ST + k * CW) // 4, CW // 4)
        cp = pltpu.make_async_copy(obuf.at[b],
                                   out_hbm.at[pl.ds(r0, CW // 4)],
                                   sem_w.at[b])
        cp.start() if start else cp.wait()

    def transpose(b, nbg):
        src = ibuf.at[b]
        dst = obuf.at[b]
        for cg in range(D // LANES):

            @plsc.parallel_loop(0, LANES, unroll=2)
            def _(r):
                cvec = ((iot + r) & (LANES - 1)) + cg * LANES
                fl0 = iot32 + cvec
                for bg in range(nbg):
                    ivec = iot + bg * LANES
                    vals = plsc.load_gather(src, [cvec, ivec])
                    fl = fl0 + bg * (LANES * D)
                    plsc.store_scatter(dst, [fl >> 7, fl & (BB - 1)],
                                       vals)

    @pl.loop(0, NCH1 // S1)
    def _(g):
        for u in range(S1):
            k = g * S1 + u

            @pl.when(g > 0)
            def _():
                write(u, k - S1, False)

            read(u, k, True)

            up = (u - G1) % S1

            @pl.when(k >= G1)
            def _():
                read(up, k - G1, False)
                transpose(up, CW // LANES)
                write(up, k - G1, True)

    for t in range(G1):
        k = NCH1 - G1 + t
        read(k % S1, k, False)
        transpose(k % S1, CW // LANES)
        write(k % S1, k, True)
    for u in range(S1):
        write(u, 0, False)

    @pl.when(w == NW - 1)
    def _():
        pltpu.sync_copy(tail_hbm, obuf.at[0, pl.ds(0, D // 2)])
        pltpu.sync_copy(obuf.at[0, pl.ds(0, D // 2)],
                        out_hbm.at[pl.ds((v - BB // 2) // 4, D // 2)])


def _tr_build(v):
    assert (NW - 1) * ST + NCH1 * CW == v - BB // 2
    mesh = plsc.VectorSubcoreMesh(core_axis_name="c", subcore_axis_name="s")
    return functools.partial(
        pl.kernel,
        out_type=jax.ShapeDtypeStruct((v // 4, 4 * D), jnp.float32),
        mesh=mesh,
        compiler_params=pltpu.CompilerParams(use_tc_tiling_on_sc=True,
                                             needs_layout_passes=False),
        scratch_types=[
            pltpu.VMEM((S1, D, CW), jnp.float32),
            pltpu.VMEM((S1, CW // 4, BB), jnp.float32),
            pltpu.SemaphoreType.DMA((S1,)),
            pltpu.SemaphoreType.DMA((S1,)),
        ],
    )(functools.partial(_tr_body, v))


def kernel(words, tags, word_table, tag_table):
    nb, l_total = words.shape
    v = word_table.shape[0]
    tail = word_table[v - BB // 2:].reshape(D // 2, 4 * D)
    wt_lin = _tr_build(v)(word_table.T, tail).reshape(v, D)
    p = _build(nb, l_total)(words.T, tags.T, wt_lin, tag_table.T)
    return p.transpose(2, 4, 0, 1, 3).reshape(nb, l_total, 2 * D)


# speedup vs baseline: 1.1680x; 1.1680x over previous
import functools

import jax
import jax.numpy as jnp
from jax import lax
from jax.experimental import pallas as pl
from jax.experimental.pallas import tpu as pltpu
from jax.experimental.pallas import tpu_sc as plsc

D = 32
NC, NS = 2, 16
NW = NC * NS
BB = 128
S = 5
G = 3
LANES = 16
TV = 1000


def _emb_body(l_total, wordsT_hbm, tagsT_hbm, wt_hbm, ttT_hbm, out_hbm,
              widx, tidx, wrows, tagv, pbuf, sem_g, sem_i, sem_w):
    w = lax.axis_index("s") * NC + lax.axis_index("c")

    pltpu.sync_copy(wordsT_hbm.at[:, pl.ds(w * BB, BB)], widx)
    pltpu.sync_copy(ttT_hbm, tagv)

    def gathers(b, l, start):
        cp = pltpu.make_async_copy(wt_hbm.at[widx.at[l]], wrows.at[b],
                                   sem_g.at[b])
        ci = pltpu.make_async_copy(tagsT_hbm.at[l, pl.ds(w * BB, BB)],
                                   tidx.at[b], sem_i.at[b])
        if start:
            cp.start()
            ci.start()
        else:
            cp.wait()
            ci.wait()

    def write(b, l, start):
        cp = pltpu.make_async_copy(pbuf.at[b], out_hbm.at[l, :, w],
                                   sem_w.at[b])
        cp.start() if start else cp.wait()

    def transpose(b, l):
        iot = lax.iota(jnp.int32, LANES)
        src = wrows.at[b]
        for cg in range(D // LANES):

            @plsc.parallel_loop(0, LANES, unroll=2)
            def _(r):
                t = (iot + r) & (LANES - 1)
                csrc = t + cg * LANES
                ct = (t >> 3) + cg * 2
                ci = t & 7
                for bg in range(BB // LANES):
                    bvec = iot + bg * LANES
                    vals = plsc.load_gather(src, [bvec, csrc])
                    plsc.store_scatter(pbuf.at[b], [ct, ci, bvec], vals)

        tvs = [tidx[b, pl.ds(bg * LANES, LANES)]
               for bg in range(BB // LANES)]

        @plsc.parallel_loop(0, D, unroll=2)
        def _(c):
            csp = jnp.full((LANES,), 0, jnp.int32) + c
            ct = (c >> 3) + 4
            ci = c & 7
            for bg in range(BB // LANES):
                vals = plsc.load_gather(tagv, [csp, tvs[bg]])
                pbuf[b, ct, ci, pl.ds(bg * LANES, LANES)] = vals

    @pl.loop(0, l_total // S)
    def _(g):
        for u in range(S):
            l = g * S + u

            @pl.when(g > 0)
            def _():
                write(u, l - S, False)

            gathers(u, l, True)

            up = (u - G) % S

            @pl.when(l >= G)
            def _():
                gathers(up, l - G, False)
                transpose(up, l - G)
                write(up, l - G, True)

    for t in range(G):
        l = l_total - G + t
        gathers(l % S, l, False)
        transpose(l % S, l)
        write(l % S, l, True)
    for u in range(S):
        write(u, 0, False)


def _build(nb, l_total):
    assert nb == NW * BB and l_total % S == 0
    mesh = plsc.VectorSubcoreMesh(core_axis_name="c", subcore_axis_name="s")
    return functools.partial(
        pl.kernel,
        out_type=jax.ShapeDtypeStruct((l_total, 2 * D // 8, NW, 8, BB),
                                      jnp.float32),
        mesh=mesh,
        compiler_params=pltpu.CompilerParams(use_tc_tiling_on_sc=False,
                                             needs_layout_passes=False),
        scratch_types=[
            pltpu.VMEM((l_total, BB), jnp.int32),
            pltpu.VMEM((S, BB), jnp.int32),
            pltpu.VMEM((S, BB, D), jnp.float32),
            pltpu.VMEM((D, TV), jnp.float32),
            pltpu.VMEM((S, 2 * D // 8, 8, BB), jnp.float32),
            pltpu.SemaphoreType.DMA((S,)),
            pltpu.SemaphoreType.DMA((S,)),
            pltpu.SemaphoreType.DMA((S,)),
        ],
    )(functools.partial(_emb_body, l_total))


ST = 31232
NCH1 = 248
S1 = 8
G1 = 6


def _tr_body(v, wt_hbm, tail_hbm, out_hbm, ibuf, obuf, sem_r, sem_w):
    w = lax.axis_index("s") * NC + lax.axis_index("c")
    iot = lax.iota(jnp.int32, LANES)
    iot32 = iot * D

    def read(b, k, start):
        i0 = pl.multiple_of(w * ST + k * BB, BB)
        cp = pltpu.make_async_copy(wt_hbm.at[:, pl.ds(i0, BB)], ibuf.at[b],
                                   sem_r.at[b])
        cp.start() if start else cp.wait()

    def write(b, k, start):
        r0 = pl.multiple_of((w * ST + k * BB) // 4, D)
        cp = pltpu.make_async_copy(obuf.at[b],
                                   out_hbm.at[pl.ds(r0, D)],
                                   sem_w.at[b])
        cp.start() if start else cp.wait()

    def transpose(b, nbg):
        src = ibuf.at[b]
        dst = obuf.at[b]
        for cg in range(D // LANES):

            @plsc.parallel_loop(0, LANES, unroll=2)
            def _(r):
                cvec = ((iot + r) & (LANES - 1)) + cg * LANES
                fl0 = iot32 + cvec
                for bg in range(nbg):
                    ivec = iot + bg * LANES
                    vals = plsc.load_gather(src, [cvec, ivec])
                    fl = fl0 + bg * (LANES * D)
                    plsc.store_scatter(dst, [fl >> 7, fl & (BB - 1)], vals)

    @pl.loop(0, NCH1 // S1)
    def _(g):
        for u in range(S1):
            k = g * S1 + u

            @pl.when(g > 0)
            def _():
                write(u, k - S1, False)

            read(u, k, True)

            up = (u - G1) % S1

            @pl.when(k >= G1)
            def _():
                read(up, k - G1, False)
                transpose(up, BB // LANES)
                write(up, k - G1, True)

    for t in range(G1):
        k = NCH1 - G1 + t
        read(k % S1, k, False)
        transpose(k % S1, BB // LANES)
        write(k % S1, k, True)
    for u in range(S1):
        write(u, 0, False)

    @pl.when(w == NW - 1)
    def _():
        pltpu.sync_copy(tail_hbm, obuf.at[0, pl.ds(0, D // 2)])
        pltpu.sync_copy(obuf.at[0, pl.ds(0, D // 2)],
                        out_hbm.at[pl.ds((v - BB // 2) // 4, D // 2)])


def _tr_build(v):
    assert (NW - 1) * ST + NCH1 * BB == v - BB // 2
    mesh = plsc.VectorSubcoreMesh(core_axis_name="c", subcore_axis_name="s")
    return functools.partial(
        pl.kernel,
        out_type=jax.ShapeDtypeStruct((v // 4, 4 * D), jnp.float32),
        mesh=mesh,
        compiler_params=pltpu.CompilerParams(use_tc_tiling_on_sc=True,
                                             needs_layout_passes=False),
        scratch_types=[
            pltpu.VMEM((S1, D, BB), jnp.float32),
            pltpu.VMEM((S1, D, BB), jnp.float32),
            pltpu.SemaphoreType.DMA((S1,)),
            pltpu.SemaphoreType.DMA((S1,)),
        ],
    )(functools.partial(_tr_body, v))


def kernel(words, tags, word_table, tag_table):
    nb, l_total = words.shape
    v = word_table.shape[0]
    tail = word_table[v - BB // 2:].reshape(D // 2, 4 * D)
    wt_lin = _tr_build(v)(word_table.T, tail).reshape(v, D)
    p = _build(nb, l_total)(words.T, tags.T, wt_lin, tag_table.T)
    return p.transpose(2, 4, 0, 1, 3).reshape(nb, l_total, 2 * D)
